# Initial kernel scaffold; baseline (speedup 1.0000x reference)
import jax
import jax.numpy as jnp
from jax.experimental import pallas as pl


def kernel(edge_index, x_train, w):
    raise NotImplementedError("write your pallas kernel here")


# speedup vs baseline: 19.8371x; 19.8371x over previous
import functools

import jax
import jax.numpy as jnp
from jax import lax
from jax.experimental import pallas as pl
from jax.experimental.pallas import tpu as pltpu
from jax.experimental.pallas import tpu_sc as plsc

_N = 2048
_D = 128
_E = 32768
_ALPHA = 0.5

_NC = 2
_NS = 16
_EDGES_PER_TILE = _E // _NS
_BAND = 512
_PASSES = 2
_ACC_WORDS = _BAND * _N
_ZW = _ACC_WORDS // _NS
_ZB = 4096
_STREAM = 128
_NSTREAMS = _EDGES_PER_TILE // _STREAM

def _sc_count_body(src_hbm, dst_hbm, out_hbm, acc, srcb, dstb, idxb, valb, zbuf, sem):
    cid = lax.axis_index("c")
    sid = lax.axis_index("s")
    ebase = sid * _EDGES_PER_TILE
    pltpu.sync_copy(src_hbm.at[pl.ds(ebase, _EDGES_PER_TILE)], srcb)
    pltpu.sync_copy(dst_hbm.at[pl.ds(ebase, _EDGES_PER_TILE)], dstb)

    @pl.loop(0, _ZB // 16)
    def _(i):
        zbuf[pl.ds(i * 16, 16)] = jnp.zeros((16,), jnp.float32)

    for p in range(_PASSES):
        band_base = cid * (_PASSES * _BAND) + p * _BAND
        zcps = [
            pltpu.async_copy(
                zbuf, acc.at[pl.ds(sid * _ZW + z * _ZB, _ZB)], sem
            )
            for z in range(_ZW // _ZB)
        ]
        for cp in zcps:
            cp.wait()
        plsc.subcore_barrier()

        @pl.loop(0, _NSTREAMS)
        def _(j):
            @pl.loop(0, _STREAM // 16)
            def _(k):
                g = j * (_STREAM // 16) + k
                s = srcb[pl.ds(g * 16, 16)]
                d = dstb[pl.ds(g * 16, 16)]
                lr = d - band_base
                m = (lr >= 0) & (lr < _BAND)
                flat = lr * _N + s
                idxb[j, pl.ds(k * 16, 16)] = jnp.where(m, flat, 0)
                valb[j, pl.ds(k * 16, 16)] = jnp.where(m, 1.0, 0.0)

        cps = [
            pltpu.async_copy(valb.at[j], acc.at[idxb.at[j]], sem, add=True)
            for j in range(_NSTREAMS)
        ]
        for cp in cps:
            cp.wait()
        plsc.subcore_barrier()

        obase = band_base * _N + sid * _ZW
        pltpu.sync_copy(acc.at[pl.ds(sid * _ZW, _ZW)], out_hbm.at[pl.ds(obase, _ZW)])


@functools.lru_cache(maxsize=1)
def _make_sc_count():
    mesh = plsc.VectorSubcoreMesh(core_axis_name="c", subcore_axis_name="s")
    return pl.kernel(
        _sc_count_body,
        out_type=jax.ShapeDtypeStruct((_N * _N,), jnp.float32),
        mesh=mesh,
        scratch_types=[
            pltpu.VMEM_SHARED((_ACC_WORDS,), jnp.float32),
            pltpu.VMEM((_EDGES_PER_TILE,), jnp.int32),
            pltpu.VMEM((_EDGES_PER_TILE,), jnp.int32),
            pltpu.VMEM((_NSTREAMS, _STREAM), jnp.int32),
            pltpu.VMEM((_NSTREAMS, _STREAM), jnp.float32),
            pltpu.VMEM((_ZB,), jnp.float32),
            pltpu.SemaphoreType.DMA,
        ],
    )


_BM = 256
_GRID = _N // _BM


def _prep_body(cnt_ref, dis_ref, vdiag_ref):
    c = cnt_ref[...]
    deg = jnp.sum(c, axis=1) + 1.0
    dis_ref[0, :] = lax.rsqrt(deg)
    odeg = jnp.sum(c, axis=0) + 1.0
    vdiag_ref[0, :] = 1.0 / odeg


_prep = pl.pallas_call(
    _prep_body,
    out_shape=(
        jax.ShapeDtypeStruct((1, _N), jnp.float32),
        jax.ShapeDtypeStruct((1, _N), jnp.float32),
    ),
)


def _rowcol_iota(i):
    rows = i * _BM + lax.broadcasted_iota(jnp.int32, (_BM, _N), 0)
    cols = lax.broadcasted_iota(jnp.int32, (_BM, _N), 1)
    return rows, cols


def _ahat_body(cnt_ref, disr_ref, disf_ref, out_ref):
    i = pl.program_id(0)
    rows, cols = _rowcol_iota(i)
    eye = jnp.where(rows == cols, 1.0, 0.0)
    drow = disr_ref[0, :]
    dcol = disf_ref[0, :]
    out_ref[...] = (drow[:, None] * (cnt_ref[...] + eye)) * dcol[None, :]


_ahat = pl.pallas_call(
    _ahat_body,
    grid=(_GRID,),
    in_specs=[
        pl.BlockSpec((_BM, _N), lambda i: (i, 0)),
        pl.BlockSpec((1, _BM), lambda i: (0, i)),
        pl.BlockSpec((1, _N), lambda i: (0, 0)),
    ],
    out_specs=pl.BlockSpec((_BM, _N), lambda i: (i, 0)),
    out_shape=jax.ShapeDtypeStruct((_N, _N), jnp.float32),
)


def _kff_body(xb_ref, xf_ref, w_ref, out_ref):
    i = pl.program_id(0)
    w = w_ref[0, :]
    xb = xb_ref[...]
    xf = xf_ref[...]
    sb = jnp.sum(xb * xb * w[None, :], axis=1)
    sf = jnp.sum(xf * xf * w[None, :], axis=1)
    cross = lax.dot_general(
        xb * w[None, :], xf, (((1,), (1,)), ((), ())),
        preferred_element_type=jnp.float32,
    )
    v = sb[:, None] + sf[None, :] - 2.0 * cross
    rows, cols = _rowcol_iota(i)
    out_ref[...] = jnp.where(rows == cols, 1.0, v)


_kff = pl.pallas_call(
    _kff_body,
    grid=(_GRID,),
    in_specs=[
        pl.BlockSpec((_BM, _D), lambda i: (i, 0)),
        pl.BlockSpec((_N, _D), lambda i: (0, 0)),
        pl.BlockSpec((1, _D), lambda i: (0, 0)),
    ],
    out_specs=pl.BlockSpec((_BM, _N), lambda i: (i, 0)),
    out_shape=jax.ShapeDtypeStruct((_N, _N), jnp.float32),
)


def _mmq_body(a_ref, b_ref, out_ref):
    i = pl.program_id(0)
    a = a_ref[...]
    s = jnp.dot(a, b_ref[...], preferred_element_type=jnp.float32)
    rows, cols = _rowcol_iota(i)
    eye = jnp.where(rows == cols, _ALPHA, 0.0)
    c1 = (1.0 - _ALPHA) * (1.0 - _ALPHA)
    c2 = _ALPHA * (1.0 - _ALPHA)
    out_ref[...] = c1 * s + c2 * a + eye


_mmq = pl.pallas_call(
    _mmq_body,
    grid=(_GRID,),
    in_specs=[
        pl.BlockSpec((_BM, _N), lambda i: (i, 0)),
        pl.BlockSpec((_N, _N), lambda i: (0, 0)),
    ],
    out_specs=pl.BlockSpec((_BM, _N), lambda i: (i, 0)),
    out_shape=jax.ShapeDtypeStruct((_N, _N), jnp.float32),
)


def _mm_body(a_ref, b_ref, out_ref):
    out_ref[...] = jnp.dot(
        a_ref[...], b_ref[...], preferred_element_type=jnp.float32
    )


_mm = pl.pallas_call(
    _mm_body,
    grid=(_GRID,),
    in_specs=[
        pl.BlockSpec((_BM, _N), lambda i: (i, 0)),
        pl.BlockSpec((_N, _N), lambda i: (0, 0)),
    ],
    out_specs=pl.BlockSpec((_BM, _N), lambda i: (i, 0)),
    out_shape=jax.ShapeDtypeStruct((_N, _N), jnp.float32),
)


def _mmt_body(t_ref, q_ref, vd_ref, out_ref):
    i = pl.program_id(0)
    o = lax.dot_general(
        t_ref[...], q_ref[...], (((1,), (1,)), ((), ())),
        preferred_element_type=jnp.float32,
    )
    rows, cols = _rowcol_iota(i)
    vd = vd_ref[0, :]
    out_ref[...] = jnp.where(rows == cols, o * vd[None, :], o)


_mmt = pl.pallas_call(
    _mmt_body,
    grid=(_GRID,),
    in_specs=[
        pl.BlockSpec((_BM, _N), lambda i: (i, 0)),
        pl.BlockSpec((_N, _N), lambda i: (0, 0)),
        pl.BlockSpec((1, _N), lambda i: (0, 0)),
    ],
    out_specs=pl.BlockSpec((_BM, _N), lambda i: (i, 0)),
    out_shape=jax.ShapeDtypeStruct((_N, _N), jnp.float32),
)


@jax.jit
def kernel(edge_index, x_train, w):
    src = edge_index[0]
    dst = edge_index[1]
    cnt = _make_sc_count()(src, dst).reshape(_N, _N)
    dis, vdiag = _prep(cnt)
    ahat = _ahat(cnt, dis, dis)
    kff = _kff(x_train, x_train, w.reshape(1, _D))
    q = _mmq(ahat, ahat)
    t = _mm(q, kff)
    return _mmt(t, q, vdiag)


# speedup vs baseline: 25.6327x; 1.2922x over previous
import functools

import jax
import jax.numpy as jnp
from jax import lax
from jax.experimental import pallas as pl
from jax.experimental.pallas import tpu as pltpu
from jax.experimental.pallas import tpu_sc as plsc

_N = 2048
_D = 128
_E = 32768
_ALPHA = 0.5

_NC = 2
_NS = 16
_EDGES_PER_TILE = _E // _NS
_BAND = 512
_PASSES = 2
_ACC_WORDS = _BAND * _N
_ZW = _ACC_WORDS // _NS
_ZB = 4096
_STREAM = 128
_NSTREAMS = _EDGES_PER_TILE // _STREAM

def _sc_count_body(
    src_hbm, dst_hbm, out_hbm, acc, srcb, dstb, idxb, valb, cbuf, zbuf, sem
):
    cid = lax.axis_index("c")
    sid = lax.axis_index("s")
    _lane_iota = lax.iota(jnp.int32, 16)
    ebase = sid * _EDGES_PER_TILE
    pltpu.sync_copy(src_hbm.at[pl.ds(ebase, _EDGES_PER_TILE)], srcb)
    pltpu.sync_copy(dst_hbm.at[pl.ds(ebase, _EDGES_PER_TILE)], dstb)

    @pl.loop(0, _ZB // 16)
    def _(i):
        zbuf[pl.ds(i * 16, 16)] = jnp.zeros((16,), jnp.float32)

    for p in range(_PASSES):
        band_base = cid * (_PASSES * _BAND) + p * _BAND
        zcps = [
            pltpu.async_copy(
                zbuf, acc.at[pl.ds(sid * _ZW + z * _ZB, _ZB)], sem
            )
            for z in range(_ZW // _ZB)
        ]
        for cp in zcps:
            cp.wait()
        plsc.subcore_barrier()

        def _compact(g, cur):
            s = srcb[pl.ds(g * 16, 16)]
            d = dstb[pl.ds(g * 16, 16)]
            lr = d - band_base
            m = (lr >= 0) & (lr < _BAND)
            flat = lr * _N + s
            plsc.store_compressed(cbuf.at[pl.ds(cur, 16)], flat, mask=m)
            return cur + jnp.sum(jnp.where(m, 1, 0))

        count = lax.fori_loop(0, _EDGES_PER_TILE // 16, _compact, 0)
        padded = lax.div(count + (_STREAM - 1), _STREAM) * _STREAM
        for z in range(_STREAM // 16):
            @pl.when(count + z * 16 < padded)
            def _():
                cbuf[pl.ds(count + z * 16, 16)] = jnp.zeros((16,), jnp.int32)

        def _fill(t, _):
            j = lax.div(t, _STREAM // 16)
            k = lax.rem(t, _STREAM // 16)
            v = cbuf[pl.ds(t * 16, 16)]
            idxb[j, pl.ds(k * 16, 16)] = v
            lane = t * 16 + _lane_iota
            valb[j, pl.ds(k * 16, 16)] = jnp.where(lane < count, 1.0, 0.0)
            return 0

        lax.fori_loop(0, lax.div(padded, 16), _fill, 0)
        nstreams = lax.div(padded, _STREAM)

        for j in range(_NSTREAMS):
            @pl.when(j < nstreams)
            def _():
                pltpu.async_copy(
                    valb.at[j], acc.at[idxb.at[j]], sem, add=True
                )
        for j in range(_NSTREAMS):
            @pl.when(j < nstreams)
            def _():
                pltpu.make_async_copy(valb.at[j], acc.at[idxb.at[j]], sem).wait()
        plsc.subcore_barrier()

        obase = band_base * _N + sid * _ZW
        pltpu.sync_copy(acc.at[pl.ds(sid * _ZW, _ZW)], out_hbm.at[pl.ds(obase, _ZW)])


@functools.lru_cache(maxsize=1)
def _make_sc_count():
    mesh = plsc.VectorSubcoreMesh(core_axis_name="c", subcore_axis_name="s")
    return pl.kernel(
        _sc_count_body,
        compiler_params=pltpu.CompilerParams(needs_layout_passes=False),
        out_type=jax.ShapeDtypeStruct((_N * _N,), jnp.float32),
        mesh=mesh,
        scratch_types=[
            pltpu.VMEM_SHARED((_ACC_WORDS,), jnp.float32),
            pltpu.VMEM((_EDGES_PER_TILE,), jnp.int32),
            pltpu.VMEM((_EDGES_PER_TILE,), jnp.int32),
            pltpu.VMEM((_NSTREAMS, _STREAM), jnp.int32),
            pltpu.VMEM((_NSTREAMS, _STREAM), jnp.float32),
            pltpu.VMEM((_EDGES_PER_TILE + 16, ), jnp.int32),
            pltpu.VMEM((_ZB,), jnp.float32),
            pltpu.SemaphoreType.DMA,
        ],
    )


_BM = 256
_GRID = _N // _BM


def _prep_body(cnt_ref, dis_ref, vdiag_ref):
    c = cnt_ref[...]
    deg = jnp.sum(c, axis=1) + 1.0
    dis_ref[0, :] = lax.rsqrt(deg)
    odeg = jnp.sum(c, axis=0) + 1.0
    vdiag_ref[0, :] = 1.0 / odeg


_prep = pl.pallas_call(
    _prep_body,
    out_shape=(
        jax.ShapeDtypeStruct((1, _N), jnp.float32),
        jax.ShapeDtypeStruct((1, _N), jnp.float32),
    ),
)


def _rowcol_iota(i):
    rows = i * _BM + lax.broadcasted_iota(jnp.int32, (_BM, _N), 0)
    cols = lax.broadcasted_iota(jnp.int32, (_BM, _N), 1)
    return rows, cols


def _ahat_body(cnt_ref, disr_ref, disf_ref, out_ref):
    i = pl.program_id(0)
    rows, cols = _rowcol_iota(i)
    eye = jnp.where(rows == cols, 1.0, 0.0)
    drow = disr_ref[0, :]
    dcol = disf_ref[0, :]
    out_ref[...] = (drow[:, None] * (cnt_ref[...] + eye)) * dcol[None, :]


_ahat = pl.pallas_call(
    _ahat_body,
    grid=(_GRID,),
    in_specs=[
        pl.BlockSpec((_BM, _N), lambda i: (i, 0)),
        pl.BlockSpec((1, _BM), lambda i: (0, i)),
        pl.BlockSpec((1, _N), lambda i: (0, 0)),
    ],
    out_specs=pl.BlockSpec((_BM, _N), lambda i: (i, 0)),
    out_shape=jax.ShapeDtypeStruct((_N, _N), jnp.float32),
)


def _kff_body(xb_ref, xf_ref, w_ref, out_ref):
    i = pl.program_id(0)
    w = w_ref[0, :]
    xb = xb_ref[...]
    xf = xf_ref[...]
    sb = jnp.sum(xb * xb * w[None, :], axis=1)
    sf = jnp.sum(xf * xf * w[None, :], axis=1)
    cross = lax.dot_general(
        xb * w[None, :], xf, (((1,), (1,)), ((), ())),
        preferred_element_type=jnp.float32,
    )
    v = sb[:, None] + sf[None, :] - 2.0 * cross
    rows, cols = _rowcol_iota(i)
    out_ref[...] = jnp.where(rows == cols, 1.0, v)


_kff = pl.pallas_call(
    _kff_body,
    grid=(_GRID,),
    in_specs=[
        pl.BlockSpec((_BM, _D), lambda i: (i, 0)),
        pl.BlockSpec((_N, _D), lambda i: (0, 0)),
        pl.BlockSpec((1, _D), lambda i: (0, 0)),
    ],
    out_specs=pl.BlockSpec((_BM, _N), lambda i: (i, 0)),
    out_shape=jax.ShapeDtypeStruct((_N, _N), jnp.float32),
)


def _mmq_body(a_ref, b_ref, out_ref):
    i = pl.program_id(0)
    a = a_ref[...]
    s = jnp.dot(a, b_ref[...], preferred_element_type=jnp.float32)
    rows, cols = _rowcol_iota(i)
    eye = jnp.where(rows == cols, _ALPHA, 0.0)
    c1 = (1.0 - _ALPHA) * (1.0 - _ALPHA)
    c2 = _ALPHA * (1.0 - _ALPHA)
    out_ref[...] = c1 * s + c2 * a + eye


_mmq = pl.pallas_call(
    _mmq_body,
    grid=(_GRID,),
    in_specs=[
        pl.BlockSpec((_BM, _N), lambda i: (i, 0)),
        pl.BlockSpec((_N, _N), lambda i: (0, 0)),
    ],
    out_specs=pl.BlockSpec((_BM, _N), lambda i: (i, 0)),
    out_shape=jax.ShapeDtypeStruct((_N, _N), jnp.float32),
)


def _mm_body(a_ref, b_ref, out_ref):
    out_ref[...] = jnp.dot(
        a_ref[...], b_ref[...], preferred_element_type=jnp.float32
    )


_mm = pl.pallas_call(
    _mm_body,
    grid=(_GRID,),
    in_specs=[
        pl.BlockSpec((_BM, _N), lambda i: (i, 0)),
        pl.BlockSpec((_N, _N), lambda i: (0, 0)),
    ],
    out_specs=pl.BlockSpec((_BM, _N), lambda i: (i, 0)),
    out_shape=jax.ShapeDtypeStruct((_N, _N), jnp.float32),
)


def _mmt_body(t_ref, q_ref, vd_ref, out_ref):
    i = pl.program_id(0)
    o = lax.dot_general(
        t_ref[...], q_ref[...], (((1,), (1,)), ((), ())),
        preferred_element_type=jnp.float32,
    )
    rows, cols = _rowcol_iota(i)
    vd = vd_ref[0, :]
    out_ref[...] = jnp.where(rows == cols, o * vd[None, :], o)


_mmt = pl.pallas_call(
    _mmt_body,
    grid=(_GRID,),
    in_specs=[
        pl.BlockSpec((_BM, _N), lambda i: (i, 0)),
        pl.BlockSpec((_N, _N), lambda i: (0, 0)),
        pl.BlockSpec((1, _N), lambda i: (0, 0)),
    ],
    out_specs=pl.BlockSpec((_BM, _N), lambda i: (i, 0)),
    out_shape=jax.ShapeDtypeStruct((_N, _N), jnp.float32),
)


@jax.jit
def kernel(edge_index, x_train, w):
    src = edge_index[0]
    dst = edge_index[1]
    cnt = _make_sc_count()(src, dst).reshape(_N, _N)
    dis, vdiag = _prep(cnt)
    ahat = _ahat(cnt, dis, dis)
    kff = _kff(x_train, x_train, w.reshape(1, _D))
    q = _mmq(ahat, ahat)
    t = _mm(q, kff)
    return _mmt(t, q, vdiag)


# speedup vs baseline: 30.2092x; 1.1785x over previous
import functools

import jax
import jax.numpy as jnp
from jax import lax
from jax.experimental import pallas as pl
from jax.experimental.pallas import tpu as pltpu
from jax.experimental.pallas import tpu_sc as plsc

_N = 2048
_D = 128
_E = 32768
_ALPHA = 0.5

_NC = 2
_NS = 16
_EDGES_PER_TILE = _E // _NS
_BAND = 512
_PASSES = 2
_ACC_WORDS = _BAND * _N
_ZW = _ACC_WORDS // _NS
_ZB = 4096
_STREAM = 128
_NSTREAMS = _EDGES_PER_TILE // _STREAM

def _sc_count_body(
    src_hbm, dst_hbm, out_hbm, acc, srcb, dstb, idxb, valb, cbuf, zbuf, sem
):
    cid = lax.axis_index("c")
    sid = lax.axis_index("s")
    _lane_iota = lax.iota(jnp.int32, 16)
    ebase = sid * _EDGES_PER_TILE
    pltpu.sync_copy(src_hbm.at[pl.ds(ebase, _EDGES_PER_TILE)], srcb)
    pltpu.sync_copy(dst_hbm.at[pl.ds(ebase, _EDGES_PER_TILE)], dstb)

    @pl.loop(0, _ZB // 16)
    def _(i):
        zbuf[pl.ds(i * 16, 16)] = jnp.zeros((16,), jnp.float32)

    for p in range(_PASSES):
        band_base = cid * (_PASSES * _BAND) + p * _BAND
        zcps = [
            pltpu.async_copy(
                zbuf, acc.at[pl.ds(sid * _ZW + z * _ZB, _ZB)], sem
            )
            for z in range(_ZW // _ZB)
        ]
        for cp in zcps:
            cp.wait()
        plsc.subcore_barrier()

        def _compact(g, cur):
            s = srcb[pl.ds(g * 16, 16)]
            d = dstb[pl.ds(g * 16, 16)]
            lr = d - band_base
            m = (lr >= 0) & (lr < _BAND)
            flat = lr * _N + s
            plsc.store_compressed(cbuf.at[pl.ds(cur, 16)], flat, mask=m)
            return cur + jnp.sum(jnp.where(m, 1, 0))

        count = lax.fori_loop(0, _EDGES_PER_TILE // 16, _compact, 0)
        padded = lax.div(count + (_STREAM - 1), _STREAM) * _STREAM
        for z in range(_STREAM // 16):
            @pl.when(count + z * 16 < padded)
            def _():
                cbuf[pl.ds(count + z * 16, 16)] = jnp.zeros((16,), jnp.int32)

        def _fill(t, _):
            j = lax.div(t, _STREAM // 16)
            k = lax.rem(t, _STREAM // 16)
            v = cbuf[pl.ds(t * 16, 16)]
            idxb[j, pl.ds(k * 16, 16)] = v
            lane = t * 16 + _lane_iota
            valb[j, pl.ds(k * 16, 16)] = jnp.where(lane < count, 1.0, 0.0)
            return 0

        lax.fori_loop(0, lax.div(padded, 16), _fill, 0)
        nstreams = lax.div(padded, _STREAM)

        for j in range(_NSTREAMS):
            @pl.when(j < nstreams)
            def _():
                pltpu.async_copy(
                    valb.at[j], acc.at[idxb.at[j]], sem, add=True
                )
        for j in range(_NSTREAMS):
            @pl.when(j < nstreams)
            def _():
                pltpu.make_async_copy(valb.at[j], acc.at[idxb.at[j]], sem).wait()
        plsc.subcore_barrier()

        obase = band_base * _N + sid * _ZW
        pltpu.sync_copy(acc.at[pl.ds(sid * _ZW, _ZW)], out_hbm.at[pl.ds(obase, _ZW)])


@functools.lru_cache(maxsize=1)
def _make_sc_count():
    mesh = plsc.VectorSubcoreMesh(core_axis_name="c", subcore_axis_name="s")
    return pl.kernel(
        _sc_count_body,
        compiler_params=pltpu.CompilerParams(needs_layout_passes=False),
        out_type=jax.ShapeDtypeStruct((_N * _N,), jnp.float32),
        mesh=mesh,
        scratch_types=[
            pltpu.VMEM_SHARED((_ACC_WORDS,), jnp.float32),
            pltpu.VMEM((_EDGES_PER_TILE,), jnp.int32),
            pltpu.VMEM((_EDGES_PER_TILE,), jnp.int32),
            pltpu.VMEM((_NSTREAMS, _STREAM), jnp.int32),
            pltpu.VMEM((_NSTREAMS, _STREAM), jnp.float32),
            pltpu.VMEM((_EDGES_PER_TILE + 16, ), jnp.int32),
            pltpu.VMEM((_ZB,), jnp.float32),
            pltpu.SemaphoreType.DMA,
        ],
    )


_BM = 256
_GRID = _N // _BM


def _rowcol_iota(i):
    rows = i * _BM + lax.broadcasted_iota(jnp.int32, (_BM, _N), 0)
    cols = lax.broadcasted_iota(jnp.int32, (_BM, _N), 1)
    return rows, cols


def _k1_body(cnt_ref, q_ref, vd_ref, ahat_scr, dis_scr):
    i = pl.program_id(0)

    @pl.when(i == 0)
    def _():
        c = cnt_ref[...]
        deg = jnp.sum(c, axis=1) + 1.0
        dis = lax.rsqrt(deg)
        odeg = jnp.sum(c, axis=0) + 1.0
        vd_ref[0, :] = 1.0 / odeg
        dis_scr[0, :] = dis
        rows = lax.broadcasted_iota(jnp.int32, (_N, _N), 0)
        cols = lax.broadcasted_iota(jnp.int32, (_N, _N), 1)
        eye = jnp.where(rows == cols, 1.0, 0.0)
        ahat_scr[...] = (dis[:, None] * (c + eye) * dis[None, :]).astype(
            jnp.bfloat16
        )

    s = jnp.dot(
        ahat_scr[pl.ds(i * _BM, _BM), :], ahat_scr[...],
        preferred_element_type=jnp.float32,
    )
    rows, cols = _rowcol_iota(i)
    eye = jnp.where(rows == cols, 1.0, 0.0)
    dis = dis_scr[0, :]
    drow = dis_scr[0, pl.ds(i * _BM, _BM)]
    aband = drow[:, None] * (cnt_ref[pl.ds(i * _BM, _BM), :] + eye) * dis[None, :]
    c1 = (1.0 - _ALPHA) * (1.0 - _ALPHA)
    c2 = _ALPHA * (1.0 - _ALPHA)
    q_ref[...] = c1 * s + c2 * aband + jnp.where(rows == cols, _ALPHA, 0.0)


_k1 = pl.pallas_call(
    _k1_body,
    grid=(_GRID,),
    in_specs=[pl.BlockSpec((_N, _N), lambda i: (0, 0))],
    out_specs=(
        pl.BlockSpec((_BM, _N), lambda i: (i, 0)),
        pl.BlockSpec((1, _N), lambda i: (0, 0)),
    ),
    out_shape=(
        jax.ShapeDtypeStruct((_N, _N), jnp.float32),
        jax.ShapeDtypeStruct((1, _N), jnp.float32),
    ),
    scratch_shapes=[
        pltpu.VMEM((_N, _N), jnp.bfloat16),
        pltpu.VMEM((1, _N), jnp.float32),
    ],
)


def _k2_body(x_ref, w_ref, q_ref, vd_ref, out_ref, kff_scr):
    i = pl.program_id(0)

    @pl.when(i == 0)
    def _():
        x = x_ref[...]
        w = w_ref[0, :]
        s = jnp.sum(x * x * w[None, :], axis=1)
        cross = lax.dot_general(
            x * w[None, :], x, (((1,), (1,)), ((), ())),
            preferred_element_type=jnp.float32,
        )
        v = s[:, None] + s[None, :] - 2.0 * cross
        rows = lax.broadcasted_iota(jnp.int32, (_N, _N), 0)
        cols = lax.broadcasted_iota(jnp.int32, (_N, _N), 1)
        kff_scr[...] = jnp.where(rows == cols, 1.0, v)

    qb = q_ref[pl.ds(i * _BM, _BM), :]
    t = jnp.dot(qb, kff_scr[...], preferred_element_type=jnp.float32)
    o = lax.dot_general(
        t, q_ref[...], (((1,), (1,)), ((), ())),
        preferred_element_type=jnp.float32,
    )
    rows, cols = _rowcol_iota(i)
    vd = vd_ref[0, :]
    out_ref[...] = jnp.where(rows == cols, o * vd[None, :], o)


_k2 = pl.pallas_call(
    _k2_body,
    grid=(_GRID,),
    in_specs=[
        pl.BlockSpec((_N, _D), lambda i: (0, 0)),
        pl.BlockSpec((1, _D), lambda i: (0, 0)),
        pl.BlockSpec((_N, _N), lambda i: (0, 0)),
        pl.BlockSpec((1, _N), lambda i: (0, 0)),
    ],
    out_specs=pl.BlockSpec((_BM, _N), lambda i: (i, 0)),
    out_shape=jax.ShapeDtypeStruct((_N, _N), jnp.float32),
    scratch_shapes=[pltpu.VMEM((_N, _N), jnp.float32)],
)


@jax.jit
def kernel(edge_index, x_train, w):
    src = edge_index[0]
    dst = edge_index[1]
    cnt = _make_sc_count()(src, dst).reshape(_N, _N)
    q, vdiag = _k1(cnt)
    return _k2(x_train, w.reshape(1, _D), q, vdiag)
